# baseline (speedup 1.0000x reference)
import functools
import jax
import jax.numpy as jnp
from jax import lax
from jax.experimental import pallas as pl
from jax.experimental.pallas import tpu as pltpu
from jax.experimental.pallas import tpu_sc as plsc

OUT = 7
N_ROIS = 512
C = 256
_OFFS = (0, 32768, 40960, 43008)
_T1 = float((56.0 * (2.0 ** 1 - 1e-6)) ** 2)
_T2 = float((56.0 * (2.0 ** 2 - 1e-6)) ** 2)
_T3 = float((56.0 * (2.0 ** 3 - 1e-6)) ** 2)

_NC = 2
_NS = 16
_R_PER_TILE = N_ROIS // (_NC * _NS)


def _bcast(v, j):
    return v.at[jnp.full((16,), j, jnp.int32)].get(mode="promise_in_bounds")


def _prologue(roisv, XI, WX, YB, HY):
    b = roisv[0, :]
    x1 = roisv[1, :]
    y1 = roisv[2, :]
    x2 = roisv[3, :]
    y2 = roisv[4, :]
    A = (x2 - x1 + 1.0) * (y2 - y1 + 1.0)
    one = jnp.int32(1)
    zero = jnp.int32(0)
    lvl = (jnp.where(A >= _T1, one, zero) + jnp.where(A >= _T2, one, zero)
           + jnp.where(A >= _T3, one, zero))
    Wl_i = lax.shift_right_logical(jnp.full((16,), 128, jnp.int32), lvl)
    scale = 0.25 / lax.shift_left(jnp.full((16,), 1, jnp.int32),
                                  lvl).astype(jnp.float32)
    off = (jnp.where(lvl >= 1, jnp.int32(_OFFS[1]), 0)
           + jnp.where(lvl >= 2, jnp.int32(_OFFS[2] - _OFFS[1]), 0)
           + jnp.where(lvl >= 3, jnp.int32(_OFFS[3] - _OFFS[2]), 0))
    base = off + b.astype(jnp.int32) * Wl_i * Wl_i
    x1s = x1 * scale
    y1s = y1 * scale
    bw2 = jnp.maximum(x2 * scale - x1s, 1.0) * (1.0 / 14.0)
    bh2 = jnp.maximum(y2 * scale - y1s, 1.0) * (1.0 / 14.0)
    lane = lax.iota(jnp.int32, 16).astype(jnp.float32) + 0.5

    def pro_body(r, _):
        Wr_i = _bcast(Wl_i, r)
        Wr_f = Wr_i.astype(jnp.float32)
        sx = _bcast(x1s, r) + lane * _bcast(bw2, r)
        sy = _bcast(y1s, r) + lane * _bcast(bh2, r)
        x = jnp.clip(sx, 0.0, Wr_f - 1.0)
        y = jnp.clip(sy, 0.0, Wr_f - 1.0)
        x0 = x.astype(jnp.int32)
        lx = x - x0.astype(jnp.float32)
        x1c = jnp.minimum(x0 + 1, Wr_i - 1)
        y0 = y.astype(jnp.int32)
        ly = y - y0.astype(jnp.float32)
        y1c = jnp.minimum(y0 + 1, Wr_i - 1)
        dxf = (x1c - x0).astype(jnp.float32)
        XI[r, 0, :] = x0
        XI[r, 1, :] = x1c - x0
        WX[r, 0, :] = (1.0 - lx + lx * (1.0 - dxf)) * 0.25
        WX[r, 1, :] = lx * dxf * 0.25
        YB[r, 0, :] = _bcast(base, r) + y0 * Wr_i
        YB[r, 1, :] = (y1c - y0) * Wr_i
        HY[r, 0, :] = 1.0 - ly
        HY[r, 1, :] = ly
        return 0

    lax.fori_loop(0, _R_PER_TILE, pro_body, 0)


def _issue(table, XI, WX, YB, HY, idx_ref, w_ref, g_ref, sem, s):
    r = s // 14
    i = s - 14 * r
    yb0 = _bcast(YB[r, 0, :], i)
    yb1 = yb0 + _bcast(YB[r, 1, :], i)
    hy_i = _bcast(HY[r, 0, :], i)
    ly_i = _bcast(HY[r, 1, :], i)
    x0v = XI[r, 0, :]
    lane = lax.iota(jnp.int32, 16)
    valid = lane < 14
    plsc.store_scatter(idx_ref, [lane], x0v + yb0, mask=valid)
    plsc.store_scatter(idx_ref, [lane + 14], x0v + yb1, mask=valid)
    hxv = WX[r, 0, :]
    lxv = WX[r, 1, :]
    w_ref[0, :] = hxv * hy_i
    w_ref[1, :] = lxv * hy_i
    w_ref[2, :] = hxv * ly_i
    w_ref[3, :] = lxv * ly_i
    pltpu.async_copy(table.at[idx_ref], g_ref, sem)


def _accum(table, out, acc, idx_ref, w_ref, g, sem, s, out_base):
    pltpu.make_async_copy(table.at[idx_ref], g, sem).wait()
    r = s // 14
    i = s - 14 * r
    oy = i // 2
    even = (i - 2 * oy) == 0
    row0 = oy * 7
    ch49 = lax.iota(jnp.int32, 16) * 49

    def make_ox_body(is_even):
        def ox_body(ox, _):
            j0 = 2 * ox
            j1 = j0 + 1
            wa = w_ref[0, :]
            wb = w_ref[1, :]
            wc = w_ref[2, :]
            wd = w_ref[3, :]
            w00a = _bcast(wa, j0)
            w01a = _bcast(wb, j0)
            w10a = _bcast(wc, j0)
            w11a = _bcast(wd, j0)
            w00b = _bcast(wa, j1)
            w01b = _bcast(wb, j1)
            w10b = _bcast(wc, j1)
            w11b = _bcast(wd, j1)
            bin_off = ch49 + (row0 + ox)

            def contrib(t):
                sl0 = pl.ds(t * 16, 16)
                sl1 = pl.ds(256 + t * 16, 16)
                return (g[j0, sl0] * w00a + g[j0, sl1] * w01a
                        + g[14 + j0, sl0] * w10a + g[14 + j0, sl1] * w11a
                        + g[j1, sl0] * w00b + g[j1, sl1] * w01b
                        + g[14 + j1, sl0] * w10b + g[14 + j1, sl1] * w11b)

            for t in range(16):
                off = bin_off + (t * 16 * 49)
                if is_even:
                    plsc.store_scatter(acc, [off], contrib(t))
                else:
                    plsc.addupdate_scatter(acc, [off], contrib(t))
            return 0
        return ox_body

    @pl.when(even)
    def _():
        lax.fori_loop(0, 7, make_ox_body(True), 0)

    @pl.when(jnp.logical_not(even))
    def _():
        lax.fori_loop(0, 7, make_ox_body(False), 0)

    @pl.when(i == 13)
    def _():
        pltpu.sync_copy(acc, out.at[out_base + r])


def _sc_body(table, roisT, out, roisv, XI, WX, YB, HY,
             idx0, idx1, idx2, idx3, w0, w1, w2, w3,
             g0, g1, g2, g3, acc, sem0, sem1, sem2, sem3):
    wid = lax.axis_index("s") * _NC + lax.axis_index("c")
    out_base = wid * _R_PER_TILE
    pltpu.sync_copy(roisT.at[wid], roisv)
    _prologue(roisv, XI, WX, YB, HY)

    n_steps = _R_PER_TILE * 14
    idxs = (idx0, idx1, idx2, idx3)
    ws = (w0, w1, w2, w3)
    gs = (g0, g1, g2, g3)
    sems = (sem0, sem1, sem2, sem3)
    for k in range(3):
        _issue(table, XI, WX, YB, HY, idxs[k], ws[k], gs[k], sems[k], k)

    def q_body(q, _):
        s_base = 4 * q
        for k in range(4):
            s = s_base + k
            _accum(table, out, acc, idxs[k], ws[k], gs[k], sems[k], s,
                   out_base)
            kn = (k + 3) % 4

            @pl.when(s + 3 < n_steps)
            def _(s=s, kn=kn):
                _issue(table, XI, WX, YB, HY, idxs[kn], ws[kn], gs[kn],
                       sems[kn], s + 3)
        return 0

    lax.fori_loop(0, n_steps // 4, q_body, 0)


@jax.jit
def kernel(feats_0, feats_1, feats_2, feats_3, rois):
    table1 = jnp.concatenate(
        [jnp.transpose(f, (0, 2, 3, 1)).reshape(-1, C)
         for f in (feats_0, feats_1, feats_2, feats_3)]
        + [jnp.zeros((1, C), jnp.float32)], axis=0)
    table = jnp.concatenate([table1[:-1], table1[1:]], axis=1)
    roisT = jnp.transpose(rois, (1, 0)).reshape(5, 32, 16).transpose(1, 0, 2)

    run = functools.partial(
        pl.kernel,
        out_type=jax.ShapeDtypeStruct((N_ROIS, C * OUT * OUT), jnp.float32),
        mesh=plsc.VectorSubcoreMesh(core_axis_name="c", subcore_axis_name="s"),
        compiler_params=pltpu.CompilerParams(needs_layout_passes=False),
        scratch_types=[
            pltpu.VMEM((5, _R_PER_TILE), jnp.float32),
            pltpu.VMEM((_R_PER_TILE, 2, 16), jnp.int32),
            pltpu.VMEM((_R_PER_TILE, 2, 16), jnp.float32),
            pltpu.VMEM((_R_PER_TILE, 2, 16), jnp.int32),
            pltpu.VMEM((_R_PER_TILE, 2, 16), jnp.float32),
        ] + [pltpu.VMEM((28,), jnp.int32)] * 4
          + [pltpu.VMEM((4, 16), jnp.float32)] * 4
          + [pltpu.VMEM((28, 2 * C), jnp.float32)] * 4
          + [pltpu.VMEM((C * OUT * OUT,), jnp.float32)]
          + [pltpu.SemaphoreType.DMA] * 4,
    )(_sc_body)
    out3 = run(table, roisT)
    return out3.reshape(N_ROIS, C, OUT, OUT)


# speedup vs baseline: 1.2518x; 1.2518x over previous
import functools
import jax
import jax.numpy as jnp
from jax import lax
from jax.experimental import pallas as pl
from jax.experimental.pallas import tpu as pltpu
from jax.experimental.pallas import tpu_sc as plsc

OUT = 7
N_ROIS = 512
C = 256
_OFFS = (0, 32768, 40960, 43008)
_T1 = float((56.0 * (2.0 ** 1 - 1e-6)) ** 2)
_T2 = float((56.0 * (2.0 ** 2 - 1e-6)) ** 2)
_T3 = float((56.0 * (2.0 ** 3 - 1e-6)) ** 2)

_NC = 2
_NS = 16
_R_PER_TILE = N_ROIS // (_NC * _NS)


def _bcast(v, j):
    return v.at[jnp.full((16,), j, jnp.int32)].get(mode="promise_in_bounds")


def _prologue(roisv, XI, WX, YB, HY):
    b = roisv[0, :]
    x1 = roisv[1, :]
    y1 = roisv[2, :]
    x2 = roisv[3, :]
    y2 = roisv[4, :]
    A = (x2 - x1 + 1.0) * (y2 - y1 + 1.0)
    one = jnp.int32(1)
    zero = jnp.int32(0)
    lvl = (jnp.where(A >= _T1, one, zero) + jnp.where(A >= _T2, one, zero)
           + jnp.where(A >= _T3, one, zero))
    Wl_i = lax.shift_right_logical(jnp.full((16,), 128, jnp.int32), lvl)
    scale = 0.25 / lax.shift_left(jnp.full((16,), 1, jnp.int32),
                                  lvl).astype(jnp.float32)
    off = (jnp.where(lvl >= 1, jnp.int32(_OFFS[1]), 0)
           + jnp.where(lvl >= 2, jnp.int32(_OFFS[2] - _OFFS[1]), 0)
           + jnp.where(lvl >= 3, jnp.int32(_OFFS[3] - _OFFS[2]), 0))
    base = off + b.astype(jnp.int32) * Wl_i * Wl_i
    x1s = x1 * scale
    y1s = y1 * scale
    bw2 = jnp.maximum(x2 * scale - x1s, 1.0) * (1.0 / 14.0)
    bh2 = jnp.maximum(y2 * scale - y1s, 1.0) * (1.0 / 14.0)
    lane = lax.iota(jnp.int32, 16).astype(jnp.float32) + 0.5

    def pro_body(r, _):
        Wr_i = _bcast(Wl_i, r)
        Wr_f = Wr_i.astype(jnp.float32)
        sx = _bcast(x1s, r) + lane * _bcast(bw2, r)
        sy = _bcast(y1s, r) + lane * _bcast(bh2, r)
        x = jnp.clip(sx, 0.0, Wr_f - 1.0)
        y = jnp.clip(sy, 0.0, Wr_f - 1.0)
        x0 = x.astype(jnp.int32)
        lx = x - x0.astype(jnp.float32)
        x1c = jnp.minimum(x0 + 1, Wr_i - 1)
        y0 = y.astype(jnp.int32)
        ly = y - y0.astype(jnp.float32)
        y1c = jnp.minimum(y0 + 1, Wr_i - 1)
        XI[r, 0, :] = x0
        XI[r, 1, :] = x1c - x0
        WX[r, 0, :] = (1.0 - lx) * 0.25
        WX[r, 1, :] = lx * 0.25
        YB[r, 0, :] = _bcast(base, r) + y0 * Wr_i
        YB[r, 1, :] = (y1c - y0) * Wr_i
        HY[r, 0, :] = 1.0 - ly
        HY[r, 1, :] = ly
        return 0

    lax.fori_loop(0, _R_PER_TILE, pro_body, 0)


def _issue(table, XI, WX, YB, HY, idx_ref, w_ref, g_ref, sem, s):
    r = s // 14
    i = s - 14 * r
    yb0 = _bcast(YB[r, 0, :], i)
    yb1 = yb0 + _bcast(YB[r, 1, :], i)
    hy_i = _bcast(HY[r, 0, :], i)
    ly_i = _bcast(HY[r, 1, :], i)
    x0v = XI[r, 0, :]
    dxv = XI[r, 1, :]
    idx_ref[pl.ds(0, 16)] = x0v + yb0
    idx_ref[pl.ds(16, 16)] = x0v + yb0 + dxv
    idx_ref[pl.ds(32, 16)] = x0v + yb1
    idx_ref[pl.ds(48, 16)] = x0v + yb1 + dxv
    hxv = WX[r, 0, :]
    lxv = WX[r, 1, :]
    w_ref[0, :] = hxv * hy_i
    w_ref[1, :] = lxv * hy_i
    w_ref[2, :] = hxv * ly_i
    w_ref[3, :] = lxv * ly_i
    pltpu.async_copy(table.at[idx_ref], g_ref, sem)


def _accum(table, out, acc, idx_ref, w_ref, g, sem, s, out_base):
    pltpu.make_async_copy(table.at[idx_ref], g, sem).wait()
    r = s // 14
    i = s - 14 * r
    oy = i // 2
    even = (i - 2 * oy) == 0
    row0 = oy * 7

    def ox_body(ox, _):
        j0 = 2 * ox
        j1 = j0 + 1
        wa = w_ref[0, :]
        wb = w_ref[1, :]
        wc = w_ref[2, :]
        wd = w_ref[3, :]
        w00a = _bcast(wa, j0)
        w01a = _bcast(wb, j0)
        w10a = _bcast(wc, j0)
        w11a = _bcast(wd, j0)
        w00b = _bcast(wa, j1)
        w01b = _bcast(wb, j1)
        w10b = _bcast(wc, j1)
        w11b = _bcast(wd, j1)

        def contrib(t):
            sl = pl.ds(t * 16, 16)
            return (g[j0, sl] * w00a + g[16 + j0, sl] * w01a
                    + g[32 + j0, sl] * w10a + g[48 + j0, sl] * w11a
                    + g[j1, sl] * w00b + g[16 + j1, sl] * w01b
                    + g[32 + j1, sl] * w10b + g[48 + j1, sl] * w11b)

        @pl.when(even)
        def _():
            for t in range(16):
                acc[row0 + ox, pl.ds(t * 16, 16)] = contrib(t)

        @pl.when(jnp.logical_not(even))
        def _():
            for t in range(16):
                sl = pl.ds(t * 16, 16)
                acc[row0 + ox, sl] = acc[row0 + ox, sl] + contrib(t)
        return 0

    lax.fori_loop(0, 7, ox_body, 0)

    @pl.when(i == 13)
    def _():
        pltpu.sync_copy(acc, out.at[out_base + r])


def _sc_body(table, roisT, out, roisv, XI, WX, YB, HY,
             idx0, idx1, w0, w1, g0, g1, acc, sem0, sem1):
    wid = lax.axis_index("s") * _NC + lax.axis_index("c")
    out_base = wid * _R_PER_TILE
    pltpu.sync_copy(roisT.at[wid], roisv)
    _prologue(roisv, XI, WX, YB, HY)

    n_steps = _R_PER_TILE * 14
    _issue(table, XI, WX, YB, HY, idx0, w0, g0, sem0, 0)

    def d_body(d, _):
        s0 = 2 * d
        s1 = s0 + 1
        _issue(table, XI, WX, YB, HY, idx1, w1, g1, sem1, s1)
        _accum(table, out, acc, idx0, w0, g0, sem0, s0, out_base)

        @pl.when(s0 + 2 < n_steps)
        def _():
            _issue(table, XI, WX, YB, HY, idx0, w0, g0, sem0, s0 + 2)

        _accum(table, out, acc, idx1, w1, g1, sem1, s1, out_base)
        return 0

    lax.fori_loop(0, n_steps // 2, d_body, 0)


@jax.jit
def kernel(feats_0, feats_1, feats_2, feats_3, rois):
    table = jnp.concatenate(
        [jnp.transpose(f, (0, 2, 3, 1)).reshape(-1, C)
         for f in (feats_0, feats_1, feats_2, feats_3)], axis=0)
    roisT = jnp.transpose(rois, (1, 0)).reshape(5, 32, 16).transpose(1, 0, 2)

    run = functools.partial(
        pl.kernel,
        out_type=jax.ShapeDtypeStruct((N_ROIS, OUT * OUT, C), jnp.float32),
        mesh=plsc.VectorSubcoreMesh(core_axis_name="c", subcore_axis_name="s"),
        scratch_types=[
            pltpu.VMEM((5, _R_PER_TILE), jnp.float32),
            pltpu.VMEM((_R_PER_TILE, 2, 16), jnp.int32),
            pltpu.VMEM((_R_PER_TILE, 2, 16), jnp.float32),
            pltpu.VMEM((_R_PER_TILE, 2, 16), jnp.int32),
            pltpu.VMEM((_R_PER_TILE, 2, 16), jnp.float32),
            pltpu.VMEM((64,), jnp.int32),
            pltpu.VMEM((64,), jnp.int32),
            pltpu.VMEM((4, 16), jnp.float32),
            pltpu.VMEM((4, 16), jnp.float32),
            pltpu.VMEM((64, C), jnp.float32),
            pltpu.VMEM((64, C), jnp.float32),
            pltpu.VMEM((OUT * OUT, C), jnp.float32),
            pltpu.SemaphoreType.DMA,
            pltpu.SemaphoreType.DMA,
        ],
    )(_sc_body)
    out3 = run(table, roisT)
    return out3.transpose(0, 2, 1).reshape(N_ROIS, C, OUT, OUT)


# speedup vs baseline: 1.2526x; 1.0007x over previous
import functools
import jax
import jax.numpy as jnp
from jax import lax
from jax.experimental import pallas as pl
from jax.experimental.pallas import tpu as pltpu
from jax.experimental.pallas import tpu_sc as plsc

OUT = 7
N_ROIS = 512
C = 256
_OFFS = (0, 32768, 40960, 43008)
_T1 = float((56.0 * (2.0 ** 1 - 1e-6)) ** 2)
_T2 = float((56.0 * (2.0 ** 2 - 1e-6)) ** 2)
_T3 = float((56.0 * (2.0 ** 3 - 1e-6)) ** 2)

_NC = 2
_NS = 16
_R_PER_TILE = N_ROIS // (_NC * _NS)


def _bcast(v, j):
    return v.at[jnp.full((16,), j, jnp.int32)].get(mode="promise_in_bounds")


def _prologue(roisv, XI, WX, YB, HY):
    b = roisv[0, :]
    x1 = roisv[1, :]
    y1 = roisv[2, :]
    x2 = roisv[3, :]
    y2 = roisv[4, :]
    A = (x2 - x1 + 1.0) * (y2 - y1 + 1.0)
    one = jnp.int32(1)
    zero = jnp.int32(0)
    lvl = (jnp.where(A >= _T1, one, zero) + jnp.where(A >= _T2, one, zero)
           + jnp.where(A >= _T3, one, zero))
    Wl_i = lax.shift_right_logical(jnp.full((16,), 128, jnp.int32), lvl)
    scale = 0.25 / lax.shift_left(jnp.full((16,), 1, jnp.int32),
                                  lvl).astype(jnp.float32)
    off = (jnp.where(lvl >= 1, jnp.int32(_OFFS[1]), 0)
           + jnp.where(lvl >= 2, jnp.int32(_OFFS[2] - _OFFS[1]), 0)
           + jnp.where(lvl >= 3, jnp.int32(_OFFS[3] - _OFFS[2]), 0))
    base = off + b.astype(jnp.int32) * Wl_i * Wl_i
    x1s = x1 * scale
    y1s = y1 * scale
    bw2 = jnp.maximum(x2 * scale - x1s, 1.0) * (1.0 / 14.0)
    bh2 = jnp.maximum(y2 * scale - y1s, 1.0) * (1.0 / 14.0)
    lane = lax.iota(jnp.int32, 16).astype(jnp.float32) + 0.5

    def pro_body(r, _):
        Wr_i = _bcast(Wl_i, r)
        Wr_f = Wr_i.astype(jnp.float32)
        sx = _bcast(x1s, r) + lane * _bcast(bw2, r)
        sy = _bcast(y1s, r) + lane * _bcast(bh2, r)
        x = jnp.clip(sx, 0.0, Wr_f - 1.0)
        y = jnp.clip(sy, 0.0, Wr_f - 1.0)
        x0 = x.astype(jnp.int32)
        lx = x - x0.astype(jnp.float32)
        x1c = jnp.minimum(x0 + 1, Wr_i - 1)
        y0 = y.astype(jnp.int32)
        ly = y - y0.astype(jnp.float32)
        y1c = jnp.minimum(y0 + 1, Wr_i - 1)
        XI[r, 0, :] = x0
        XI[r, 1, :] = x1c - x0
        WX[r, 0, :] = (1.0 - lx) * 0.25
        WX[r, 1, :] = lx * 0.25
        YB[r, 0, :] = _bcast(base, r) + y0 * Wr_i
        YB[r, 1, :] = (y1c - y0) * Wr_i
        HY[r, 0, :] = 1.0 - ly
        HY[r, 1, :] = ly
        return 0

    lax.fori_loop(0, _R_PER_TILE, pro_body, 0)


def _issue(table, XI, WX, YB, HY, idx_ref, w_ref, g_ref, sem, s):
    r = s // 14
    i = s - 14 * r
    yb0 = _bcast(YB[r, 0, :], i)
    yb1 = yb0 + _bcast(YB[r, 1, :], i)
    hy_i = _bcast(HY[r, 0, :], i)
    ly_i = _bcast(HY[r, 1, :], i)
    x0v = XI[r, 0, :]
    dxv = XI[r, 1, :]
    idx_ref[pl.ds(0, 16)] = x0v + yb0
    idx_ref[pl.ds(16, 16)] = x0v + yb0 + dxv
    idx_ref[pl.ds(32, 16)] = x0v + yb1
    idx_ref[pl.ds(48, 16)] = x0v + yb1 + dxv
    hxv = WX[r, 0, :]
    lxv = WX[r, 1, :]
    w_ref[0, :] = hxv * hy_i
    w_ref[1, :] = lxv * hy_i
    w_ref[2, :] = hxv * ly_i
    w_ref[3, :] = lxv * ly_i
    pltpu.async_copy(table.at[idx_ref], g_ref, sem)


def _accum(table, out, acc, idx_ref, w_ref, g, sem, s, out_base):
    pltpu.make_async_copy(table.at[idx_ref], g, sem).wait()
    r = s // 14
    i = s - 14 * r
    oy = i // 2
    even = (i - 2 * oy) == 0
    row0 = oy * 7

    def ox_body(ox, _):
        j0 = 2 * ox
        j1 = j0 + 1
        wa = w_ref[0, :]
        wb = w_ref[1, :]
        wc = w_ref[2, :]
        wd = w_ref[3, :]
        w00a = _bcast(wa, j0)
        w01a = _bcast(wb, j0)
        w10a = _bcast(wc, j0)
        w11a = _bcast(wd, j0)
        w00b = _bcast(wa, j1)
        w01b = _bcast(wb, j1)
        w10b = _bcast(wc, j1)
        w11b = _bcast(wd, j1)

        def contrib(t):
            sl = pl.ds(t * 16, 16)
            return (g[j0, sl] * w00a + g[16 + j0, sl] * w01a
                    + g[32 + j0, sl] * w10a + g[48 + j0, sl] * w11a
                    + g[j1, sl] * w00b + g[16 + j1, sl] * w01b
                    + g[32 + j1, sl] * w10b + g[48 + j1, sl] * w11b)

        @pl.when(even)
        def _():
            for t in range(16):
                acc[row0 + ox, pl.ds(t * 16, 16)] = contrib(t)

        @pl.when(jnp.logical_not(even))
        def _():
            for t in range(16):
                sl = pl.ds(t * 16, 16)
                acc[row0 + ox, sl] = acc[row0 + ox, sl] + contrib(t)
        return 0

    lax.fori_loop(0, 7, ox_body, 0)

    @pl.when(i == 13)
    def _():
        pltpu.sync_copy(acc, out.at[out_base + r])


def _sc_body(table, roisT, out, roisv, XI, WX, YB, HY,
             idx0, idx1, w0, w1, g0, g1, acc, sem0, sem1):
    wid = lax.axis_index("s") * _NC + lax.axis_index("c")
    out_base = wid * _R_PER_TILE
    pltpu.sync_copy(roisT.at[wid], roisv)
    _prologue(roisv, XI, WX, YB, HY)

    n_steps = _R_PER_TILE * 14
    _issue(table, XI, WX, YB, HY, idx0, w0, g0, sem0, 0)

    def d_body(d, _):
        s0 = 2 * d
        s1 = s0 + 1
        _issue(table, XI, WX, YB, HY, idx1, w1, g1, sem1, s1)
        _accum(table, out, acc, idx0, w0, g0, sem0, s0, out_base)

        @pl.when(s0 + 2 < n_steps)
        def _():
            _issue(table, XI, WX, YB, HY, idx0, w0, g0, sem0, s0 + 2)

        _accum(table, out, acc, idx1, w1, g1, sem1, s1, out_base)
        return 0

    lax.fori_loop(0, n_steps // 2, d_body, 0)


@jax.jit
def kernel(feats_0, feats_1, feats_2, feats_3, rois):
    table = jnp.concatenate(
        [jnp.transpose(f, (0, 2, 3, 1)).reshape(-1, C)
         for f in (feats_0, feats_1, feats_2, feats_3)], axis=0)
    roisT = jnp.transpose(rois, (1, 0)).reshape(5, 32, 16).transpose(1, 0, 2)

    run = functools.partial(
        pl.kernel,
        out_type=jax.ShapeDtypeStruct((N_ROIS, OUT * OUT, C), jnp.float32),
        mesh=plsc.VectorSubcoreMesh(core_axis_name="c", subcore_axis_name="s"),
        compiler_params=pltpu.CompilerParams(needs_layout_passes=False),
        scratch_types=[
            pltpu.VMEM((5, _R_PER_TILE), jnp.float32),
            pltpu.VMEM((_R_PER_TILE, 2, 16), jnp.int32),
            pltpu.VMEM((_R_PER_TILE, 2, 16), jnp.float32),
            pltpu.VMEM((_R_PER_TILE, 2, 16), jnp.int32),
            pltpu.VMEM((_R_PER_TILE, 2, 16), jnp.float32),
            pltpu.VMEM((64,), jnp.int32),
            pltpu.VMEM((64,), jnp.int32),
            pltpu.VMEM((4, 16), jnp.float32),
            pltpu.VMEM((4, 16), jnp.float32),
            pltpu.VMEM((64, C), jnp.float32),
            pltpu.VMEM((64, C), jnp.float32),
            pltpu.VMEM((OUT * OUT, C), jnp.float32),
            pltpu.SemaphoreType.DMA,
            pltpu.SemaphoreType.DMA,
        ],
    )(_sc_body)
    out3 = run(table, roisT)
    return out3.transpose(0, 2, 1).reshape(N_ROIS, C, OUT, OUT)


# speedup vs baseline: 1.2757x; 1.0184x over previous
import functools
import jax
import jax.numpy as jnp
from jax import lax
from jax.experimental import pallas as pl
from jax.experimental.pallas import tpu as pltpu
from jax.experimental.pallas import tpu_sc as plsc

OUT = 7
N_ROIS = 512
C = 256
_OFFS = (0, 32768, 40960, 43008)
_T1 = float((56.0 * (2.0 ** 1 - 1e-6)) ** 2)
_T2 = float((56.0 * (2.0 ** 2 - 1e-6)) ** 2)
_T3 = float((56.0 * (2.0 ** 3 - 1e-6)) ** 2)

_NC = 2
_NS = 16
_R_PER_TILE = N_ROIS // (_NC * _NS)


def _bcast(v, j):
    return v.at[jnp.full((16,), j, jnp.int32)].get(mode="promise_in_bounds")


def _prologue(roisv, XI, WX, YB, HY):
    b = roisv[0, :]
    x1 = roisv[1, :]
    y1 = roisv[2, :]
    x2 = roisv[3, :]
    y2 = roisv[4, :]
    A = (x2 - x1 + 1.0) * (y2 - y1 + 1.0)
    one = jnp.int32(1)
    zero = jnp.int32(0)
    lvl = (jnp.where(A >= _T1, one, zero) + jnp.where(A >= _T2, one, zero)
           + jnp.where(A >= _T3, one, zero))
    Wl_i = lax.shift_right_logical(jnp.full((16,), 128, jnp.int32), lvl)
    scale = 0.25 / lax.shift_left(jnp.full((16,), 1, jnp.int32),
                                  lvl).astype(jnp.float32)
    off = (jnp.where(lvl >= 1, jnp.int32(_OFFS[1]), 0)
           + jnp.where(lvl >= 2, jnp.int32(_OFFS[2] - _OFFS[1]), 0)
           + jnp.where(lvl >= 3, jnp.int32(_OFFS[3] - _OFFS[2]), 0))
    base = off + b.astype(jnp.int32) * Wl_i * Wl_i
    x1s = x1 * scale
    y1s = y1 * scale
    bw2 = jnp.maximum(x2 * scale - x1s, 1.0) * (1.0 / 14.0)
    bh2 = jnp.maximum(y2 * scale - y1s, 1.0) * (1.0 / 14.0)
    lane = lax.iota(jnp.int32, 16).astype(jnp.float32) + 0.5

    def pro_body(r, _):
        Wr_i = _bcast(Wl_i, r)
        Wr_f = Wr_i.astype(jnp.float32)
        sx = _bcast(x1s, r) + lane * _bcast(bw2, r)
        sy = _bcast(y1s, r) + lane * _bcast(bh2, r)
        x = jnp.clip(sx, 0.0, Wr_f - 1.0)
        y = jnp.clip(sy, 0.0, Wr_f - 1.0)
        x0 = x.astype(jnp.int32)
        lx = x - x0.astype(jnp.float32)
        x1c = jnp.minimum(x0 + 1, Wr_i - 1)
        y0 = y.astype(jnp.int32)
        ly = y - y0.astype(jnp.float32)
        y1c = jnp.minimum(y0 + 1, Wr_i - 1)
        XI[r, 0, :] = x0
        XI[r, 1, :] = x1c - x0
        WX[r, 0, :] = (1.0 - lx) * 0.25
        WX[r, 1, :] = lx * 0.25
        YB[r, 0, :] = _bcast(base, r) + y0 * Wr_i
        YB[r, 1, :] = (y1c - y0) * Wr_i
        HY[r, 0, :] = 1.0 - ly
        HY[r, 1, :] = ly
        return 0

    lax.fori_loop(0, _R_PER_TILE, pro_body, 0)


def _issue(table, XI, WX, YB, HY, idx_ref, w_ref, g_ref, sem, s):
    r = s // 14
    i = s - 14 * r
    yb0 = _bcast(YB[r, 0, :], i)
    yb1 = yb0 + _bcast(YB[r, 1, :], i)
    hy_i = _bcast(HY[r, 0, :], i)
    ly_i = _bcast(HY[r, 1, :], i)
    x0v = XI[r, 0, :]
    dxv = XI[r, 1, :]
    lane = lax.iota(jnp.int32, 16)
    valid = lane < 14
    plsc.store_scatter(idx_ref, [lane], x0v + yb0, mask=valid)
    plsc.store_scatter(idx_ref, [lane + 14], x0v + yb0 + dxv, mask=valid)
    plsc.store_scatter(idx_ref, [lane + 28], x0v + yb1, mask=valid)
    plsc.store_scatter(idx_ref, [lane + 42], x0v + yb1 + dxv, mask=valid)
    hxv = WX[r, 0, :]
    lxv = WX[r, 1, :]
    w_ref[0, :] = hxv * hy_i
    w_ref[1, :] = lxv * hy_i
    w_ref[2, :] = hxv * ly_i
    w_ref[3, :] = lxv * ly_i
    pltpu.async_copy(table.at[idx_ref], g_ref, sem)


def _accum(table, out, acc, idx_ref, w_ref, g, sem, s, out_base):
    pltpu.make_async_copy(table.at[idx_ref], g, sem).wait()
    r = s // 14
    i = s - 14 * r
    oy = i // 2
    even = (i - 2 * oy) == 0
    row0 = oy * 7

    def ox_body(ox, _):
        j0 = 2 * ox
        j1 = j0 + 1
        wa = w_ref[0, :]
        wb = w_ref[1, :]
        wc = w_ref[2, :]
        wd = w_ref[3, :]
        w00a = _bcast(wa, j0)
        w01a = _bcast(wb, j0)
        w10a = _bcast(wc, j0)
        w11a = _bcast(wd, j0)
        w00b = _bcast(wa, j1)
        w01b = _bcast(wb, j1)
        w10b = _bcast(wc, j1)
        w11b = _bcast(wd, j1)

        def contrib(t):
            sl = pl.ds(t * 16, 16)
            return (g[j0, sl] * w00a + g[14 + j0, sl] * w01a
                    + g[28 + j0, sl] * w10a + g[42 + j0, sl] * w11a
                    + g[j1, sl] * w00b + g[14 + j1, sl] * w01b
                    + g[28 + j1, sl] * w10b + g[42 + j1, sl] * w11b)

        @pl.when(even)
        def _():
            for t in range(16):
                acc[row0 + ox, pl.ds(t * 16, 16)] = contrib(t)

        @pl.when(jnp.logical_not(even))
        def _():
            for t in range(16):
                sl = pl.ds(t * 16, 16)
                acc[row0 + ox, sl] = acc[row0 + ox, sl] + contrib(t)
        return 0

    lax.fori_loop(0, 7, ox_body, 0)

    @pl.when(i == 13)
    def _():
        pltpu.sync_copy(acc, out.at[out_base + r])


def _sc_body(table, roisT, out, roisv, XI, WX, YB, HY,
             idx0, idx1, w0, w1, g0, g1, acc, sem0, sem1):
    wid = lax.axis_index("s") * _NC + lax.axis_index("c")
    out_base = wid * _R_PER_TILE
    pltpu.sync_copy(roisT.at[wid], roisv)
    _prologue(roisv, XI, WX, YB, HY)

    n_steps = _R_PER_TILE * 14
    _issue(table, XI, WX, YB, HY, idx0, w0, g0, sem0, 0)

    def d_body(d, _):
        s0 = 2 * d
        s1 = s0 + 1
        _issue(table, XI, WX, YB, HY, idx1, w1, g1, sem1, s1)
        _accum(table, out, acc, idx0, w0, g0, sem0, s0, out_base)

        @pl.when(s0 + 2 < n_steps)
        def _():
            _issue(table, XI, WX, YB, HY, idx0, w0, g0, sem0, s0 + 2)

        _accum(table, out, acc, idx1, w1, g1, sem1, s1, out_base)
        return 0

    lax.fori_loop(0, n_steps // 2, d_body, 0)


@jax.jit
def kernel(feats_0, feats_1, feats_2, feats_3, rois):
    table = jnp.concatenate(
        [jnp.transpose(f, (0, 2, 3, 1)).reshape(-1, C)
         for f in (feats_0, feats_1, feats_2, feats_3)], axis=0)
    roisT = jnp.transpose(rois, (1, 0)).reshape(5, 32, 16).transpose(1, 0, 2)

    run = functools.partial(
        pl.kernel,
        out_type=jax.ShapeDtypeStruct((N_ROIS, OUT * OUT, C), jnp.float32),
        mesh=plsc.VectorSubcoreMesh(core_axis_name="c", subcore_axis_name="s"),
        compiler_params=pltpu.CompilerParams(needs_layout_passes=False),
        scratch_types=[
            pltpu.VMEM((5, _R_PER_TILE), jnp.float32),
            pltpu.VMEM((_R_PER_TILE, 2, 16), jnp.int32),
            pltpu.VMEM((_R_PER_TILE, 2, 16), jnp.float32),
            pltpu.VMEM((_R_PER_TILE, 2, 16), jnp.int32),
            pltpu.VMEM((_R_PER_TILE, 2, 16), jnp.float32),
            pltpu.VMEM((56,), jnp.int32),
            pltpu.VMEM((56,), jnp.int32),
            pltpu.VMEM((4, 16), jnp.float32),
            pltpu.VMEM((4, 16), jnp.float32),
            pltpu.VMEM((56, C), jnp.float32),
            pltpu.VMEM((56, C), jnp.float32),
            pltpu.VMEM((OUT * OUT, C), jnp.float32),
            pltpu.SemaphoreType.DMA,
            pltpu.SemaphoreType.DMA,
        ],
    )(_sc_body)
    out3 = run(table, roisT)
    return out3.transpose(0, 2, 1).reshape(N_ROIS, C, OUT, OUT)


# speedup vs baseline: 1.3646x; 1.0697x over previous
import functools
import jax
import jax.numpy as jnp
from jax import lax
from jax.experimental import pallas as pl
from jax.experimental.pallas import tpu as pltpu
from jax.experimental.pallas import tpu_sc as plsc

OUT = 7
N_ROIS = 512
C = 256
_OFFS = (0, 32768, 40960, 43008)
_T1 = float((56.0 * (2.0 ** 1 - 1e-6)) ** 2)
_T2 = float((56.0 * (2.0 ** 2 - 1e-6)) ** 2)
_T3 = float((56.0 * (2.0 ** 3 - 1e-6)) ** 2)

_NC = 2
_NS = 16
_R_PER_TILE = N_ROIS // (_NC * _NS)


def _bcast(v, j):
    return v.at[jnp.full((16,), j, jnp.int32)].get(mode="promise_in_bounds")


def _prologue(roisv, XI, WX, YB, HY):
    b = roisv[0, :]
    x1 = roisv[1, :]
    y1 = roisv[2, :]
    x2 = roisv[3, :]
    y2 = roisv[4, :]
    A = (x2 - x1 + 1.0) * (y2 - y1 + 1.0)
    one = jnp.int32(1)
    zero = jnp.int32(0)
    lvl = (jnp.where(A >= _T1, one, zero) + jnp.where(A >= _T2, one, zero)
           + jnp.where(A >= _T3, one, zero))
    Wl_i = lax.shift_right_logical(jnp.full((16,), 128, jnp.int32), lvl)
    scale = 0.25 / lax.shift_left(jnp.full((16,), 1, jnp.int32),
                                  lvl).astype(jnp.float32)
    off = (jnp.where(lvl >= 1, jnp.int32(_OFFS[1]), 0)
           + jnp.where(lvl >= 2, jnp.int32(_OFFS[2] - _OFFS[1]), 0)
           + jnp.where(lvl >= 3, jnp.int32(_OFFS[3] - _OFFS[2]), 0))
    base = off + b.astype(jnp.int32) * Wl_i * Wl_i
    x1s = x1 * scale
    y1s = y1 * scale
    bw2 = jnp.maximum(x2 * scale - x1s, 1.0) * (1.0 / 14.0)
    bh2 = jnp.maximum(y2 * scale - y1s, 1.0) * (1.0 / 14.0)
    lane = lax.iota(jnp.int32, 16).astype(jnp.float32) + 0.5

    def pro_body(r, _):
        Wr_i = _bcast(Wl_i, r)
        Wr_f = Wr_i.astype(jnp.float32)
        sx = _bcast(x1s, r) + lane * _bcast(bw2, r)
        sy = _bcast(y1s, r) + lane * _bcast(bh2, r)
        x = jnp.clip(sx, 0.0, Wr_f - 1.0)
        y = jnp.clip(sy, 0.0, Wr_f - 1.0)
        x0 = x.astype(jnp.int32)
        lx = x - x0.astype(jnp.float32)
        x1c = jnp.minimum(x0 + 1, Wr_i - 1)
        y0 = y.astype(jnp.int32)
        ly = y - y0.astype(jnp.float32)
        y1c = jnp.minimum(y0 + 1, Wr_i - 1)
        XI[r, 0, :] = x0
        XI[r, 1, :] = x1c - x0
        WX[r, 0, :] = (1.0 - lx) * 0.25
        WX[r, 1, :] = lx * 0.25
        YB[r, 0, :] = _bcast(base, r) + y0 * Wr_i
        YB[r, 1, :] = (y1c - y0) * Wr_i
        HY[r, 0, :] = 1.0 - ly
        HY[r, 1, :] = ly
        return 0

    lax.fori_loop(0, _R_PER_TILE, pro_body, 0)


def _issue(table, XI, WX, YB, HY, idx_ref, w_ref, g_ref, sem, s):
    r = s // 14
    i = s - 14 * r
    yb0 = _bcast(YB[r, 0, :], i)
    yb1 = yb0 + _bcast(YB[r, 1, :], i)
    hy_i = _bcast(HY[r, 0, :], i)
    ly_i = _bcast(HY[r, 1, :], i)
    x0v = XI[r, 0, :]
    dxv = XI[r, 1, :]
    lane = lax.iota(jnp.int32, 16)
    valid = lane < 14
    plsc.store_scatter(idx_ref, [lane], x0v + yb0, mask=valid)
    plsc.store_scatter(idx_ref, [lane + 14], x0v + yb0 + dxv, mask=valid)
    plsc.store_scatter(idx_ref, [lane + 28], x0v + yb1, mask=valid)
    plsc.store_scatter(idx_ref, [lane + 42], x0v + yb1 + dxv, mask=valid)
    hxv = WX[r, 0, :]
    lxv = WX[r, 1, :]
    w_ref[0, :] = hxv * hy_i
    w_ref[1, :] = lxv * hy_i
    w_ref[2, :] = hxv * ly_i
    w_ref[3, :] = lxv * ly_i
    pltpu.async_copy(table.at[idx_ref], g_ref, sem)


def _accum(table, out, acc, idx_ref, w_ref, g, sem, s, out_base):
    pltpu.make_async_copy(table.at[idx_ref], g, sem).wait()
    r = s // 14
    i = s - 14 * r
    oy = i // 2
    even = (i - 2 * oy) == 0
    row0 = oy * 7

    def ox_body(ox, _):
        j0 = 2 * ox
        j1 = j0 + 1
        wa = w_ref[0, :]
        wb = w_ref[1, :]
        wc = w_ref[2, :]
        wd = w_ref[3, :]
        w00a = _bcast(wa, j0)
        w01a = _bcast(wb, j0)
        w10a = _bcast(wc, j0)
        w11a = _bcast(wd, j0)
        w00b = _bcast(wa, j1)
        w01b = _bcast(wb, j1)
        w10b = _bcast(wc, j1)
        w11b = _bcast(wd, j1)

        rows_ws = ((j0, w00a), (14 + j0, w01a), (28 + j0, w10a),
                   (42 + j0, w11a), (j1, w00b), (14 + j1, w01b),
                   (28 + j1, w10b), (42 + j1, w11b))

        def contrib(t):
            va = None
            vb = None
            for row, w in rows_ws:
                x32 = plsc.bitcast(g[row, pl.ds(t * 16, 16)], jnp.bfloat16)
                xa, xb = plsc.unpack(x32,
                                     format=plsc.PackFormat.INTERLEAVED)
                va = xa * w if va is None else va + xa * w
                vb = xb * w if vb is None else vb + xb * w
            return va, vb

        @pl.when(even)
        def _():
            for t in range(8):
                va, vb = contrib(t)
                acc[row0 + ox, pl.ds(t * 32, 16)] = va
                acc[row0 + ox, pl.ds(t * 32 + 16, 16)] = vb

        @pl.when(jnp.logical_not(even))
        def _():
            for t in range(8):
                va, vb = contrib(t)
                sla = pl.ds(t * 32, 16)
                slb = pl.ds(t * 32 + 16, 16)
                acc[row0 + ox, sla] = acc[row0 + ox, sla] + va
                acc[row0 + ox, slb] = acc[row0 + ox, slb] + vb
        return 0

    lax.fori_loop(0, 7, ox_body, 0)

    @pl.when(i == 13)
    def _():
        pltpu.sync_copy(acc, out.at[out_base + r])


def _sc_body(table, roisT, out, roisv, XI, WX, YB, HY,
             idx0, idx1, w0, w1, g0, g1, acc, sem0, sem1):
    wid = lax.axis_index("s") * _NC + lax.axis_index("c")
    out_base = wid * _R_PER_TILE
    pltpu.sync_copy(roisT.at[wid], roisv)
    _prologue(roisv, XI, WX, YB, HY)

    n_steps = _R_PER_TILE * 14
    _issue(table, XI, WX, YB, HY, idx0, w0, g0, sem0, 0)

    def d_body(d, _):
        s0 = 2 * d
        s1 = s0 + 1
        _issue(table, XI, WX, YB, HY, idx1, w1, g1, sem1, s1)
        _accum(table, out, acc, idx0, w0, g0, sem0, s0, out_base)

        @pl.when(s0 + 2 < n_steps)
        def _():
            _issue(table, XI, WX, YB, HY, idx0, w0, g0, sem0, s0 + 2)

        _accum(table, out, acc, idx1, w1, g1, sem1, s1, out_base)
        return 0

    lax.fori_loop(0, n_steps // 2, d_body, 0)


@jax.jit
def kernel(feats_0, feats_1, feats_2, feats_3, rois):
    table = jnp.concatenate(
        [jnp.transpose(f, (0, 2, 3, 1)).reshape(-1, C)
         for f in (feats_0, feats_1, feats_2, feats_3)], axis=0)
    table = (table.reshape(-1, 8, 2, 16).swapaxes(2, 3).reshape(-1, C)
             .astype(jnp.bfloat16))
    table = lax.bitcast_convert_type(table.reshape(-1, C // 2, 2), jnp.int32)
    roisT = jnp.transpose(rois, (1, 0)).reshape(5, 32, 16).transpose(1, 0, 2)

    run = functools.partial(
        pl.kernel,
        out_type=jax.ShapeDtypeStruct((N_ROIS, OUT * OUT, C), jnp.float32),
        mesh=plsc.VectorSubcoreMesh(core_axis_name="c", subcore_axis_name="s"),
        compiler_params=pltpu.CompilerParams(needs_layout_passes=False),
        scratch_types=[
            pltpu.VMEM((5, _R_PER_TILE), jnp.float32),
            pltpu.VMEM((_R_PER_TILE, 2, 16), jnp.int32),
            pltpu.VMEM((_R_PER_TILE, 2, 16), jnp.float32),
            pltpu.VMEM((_R_PER_TILE, 2, 16), jnp.int32),
            pltpu.VMEM((_R_PER_TILE, 2, 16), jnp.float32),
            pltpu.VMEM((56,), jnp.int32),
            pltpu.VMEM((56,), jnp.int32),
            pltpu.VMEM((4, 16), jnp.float32),
            pltpu.VMEM((4, 16), jnp.float32),
            pltpu.VMEM((56, C // 2), jnp.int32),
            pltpu.VMEM((56, C // 2), jnp.int32),
            pltpu.VMEM((OUT * OUT, C), jnp.float32),
            pltpu.SemaphoreType.DMA,
            pltpu.SemaphoreType.DMA,
        ],
    )(_sc_body)
    out3 = run(table, roisT)
    return out3.transpose(0, 2, 1).reshape(N_ROIS, C, OUT, OUT)
